# baseline (speedup 1.0000x reference)
import functools

import jax
import jax.numpy as jnp
from jax import lax
from jax.experimental import pallas as pl
from jax.experimental.pallas import tpu as pltpu
from jax.experimental.pallas import tpu_sc as plsc

NUM_FIELDS = 26
FE_DIM = 512
F2 = NUM_FIELDS * NUM_FIELDS
FPAD = 768
VOCAB = 100000
B = 4096

NUM_WORKERS = 32
IDX_PER_W = B // NUM_WORKERS
LANES = 16


def _mlp_kernel(x_ref, wc1_ref, c1_ref, d2_ref, c2_ref, d3_ref, c3_ref, out_ref):
    h1 = jnp.dot(x_ref[...], wc1_ref[...], preferred_element_type=jnp.float32)
    h1 = jnp.maximum(h1 + c1_ref[...], 0.0)
    h2 = jnp.dot(h1, d2_ref[...], preferred_element_type=jnp.float32)
    h2 = jnp.maximum(h2 + c2_ref[...], 0.0)
    m2 = jnp.sum(h2, axis=0, keepdims=True) * (1.0 / x_ref.shape[0])
    out_ref[...] = (
        jnp.dot(m2, d3_ref[...], preferred_element_type=jnp.float32) + c3_ref[...]
    )


def _sc_scatter(table_ref, idx_hbm, gvec_hbm, idx_v, gv_v, row_v, sem):
    wid = lax.axis_index("s") * 2 + lax.axis_index("c")
    pltpu.sync_copy(idx_hbm.at[pl.ds(wid * IDX_PER_W, IDX_PER_W)], idx_v)
    pltpu.sync_copy(gvec_hbm, gv_v)

    @pl.loop(0, IDX_PER_W)
    def _fill(j):
        for k in range(FPAD // LANES):
            row_v[j, pl.ds(k * LANES, LANES)] = gv_v[pl.ds(k * LANES, LANES)]

    pltpu.async_copy(row_v, table_ref.at[idx_v], sem).wait()


def _scatter_pallas(table, indexes, gvec_pad):
    mesh = plsc.VectorSubcoreMesh(core_axis_name="c", subcore_axis_name="s")

    run = pl.kernel(
        _sc_scatter,
        out_type=(),
        mesh=mesh,
        scratch_types=[
            pltpu.VMEM((IDX_PER_W,), jnp.int32),
            pltpu.VMEM((FPAD,), jnp.float32),
            pltpu.VMEM((IDX_PER_W, FPAD), jnp.float32),
            pltpu.SemaphoreType.DMA,
        ],
        compiler_params=pltpu.CompilerParams(needs_layout_passes=False),
    )
    run(table, indexes, gvec_pad)


def kernel(feature_emb, indexes, graph_dict, W1, b1, W2, b2, W3, b3):
    f = NUM_FIELDS
    Wc1 = W1[:, :FE_DIM, :].transpose(1, 0, 2).reshape(FE_DIM, F2)
    diag = W1[jnp.arange(f), FE_DIM + jnp.arange(f), :]
    c1 = (diag + b1).reshape(1, F2)
    eye = jnp.eye(f, dtype=W2.dtype)
    D2 = (W2[:, :, None, :] * eye[:, None, :, None]).reshape(F2, F2)
    D3 = (W3[:, :, None, :] * eye[:, None, :, None]).reshape(F2, F2)
    c2 = b2.reshape(1, F2)
    c3 = b3.reshape(1, F2)

    gvec = pl.pallas_call(
        _mlp_kernel,
        out_shape=jax.ShapeDtypeStruct((1, F2), jnp.float32),
    )(feature_emb, Wc1, c1, D2, c2, D3, c3)
    gvec_pad = jnp.pad(gvec.reshape(F2), (0, FPAD - F2))

    padded = jnp.pad(graph_dict, ((0, 0), (0, FPAD - F2)))
    table = jax.new_ref(padded)
    _scatter_pallas(table, indexes.astype(jnp.int32), gvec_pad)
    return table[...][:, :F2]
